# baseline (speedup 1.0000x reference)
import functools

import jax
import jax.numpy as jnp
from jax import lax
from jax.experimental import pallas as pl
from jax.experimental.pallas import tpu as pltpu
from jax.experimental.pallas import tpu_sc as plsc

BATCH = 16384
EMB_K = 64
NUM_CORES = 2
NUM_SUBCORES = 16
NW = NUM_CORES * NUM_SUBCORES
B_PER_W = BATCH // NW
NBANDS = EMB_K // 8
NPAIR = BATCH // 2


def _sc_gather_body(idx_hbm, tbl_hbm, pair_out, idx_v, rows_v, sem):
    wid = lax.axis_index("s") * NUM_CORES + lax.axis_index("c")
    base = wid * B_PER_W
    pltpu.sync_copy(idx_hbm.at[pl.ds(base, B_PER_W)], idx_v)
    cp = pltpu.async_copy(tbl_hbm.at[idx_v], rows_v, sem)
    row0 = 512 * (wid // 2)
    col0 = EMB_K * (wid % 2)
    cp.wait()
    pltpu.sync_copy(rows_v, pair_out.at[pl.ds(row0, B_PER_W), pl.ds(col0, EMB_K)])


@functools.cache
def _sc_gather():
    return pl.kernel(
        _sc_gather_body,
        mesh=plsc.VectorSubcoreMesh(
            core_axis_name="c", subcore_axis_name="s",
            num_cores=NUM_CORES, num_subcores=NUM_SUBCORES),
        out_type=jax.ShapeDtypeStruct((NPAIR, 128), jnp.float32),
        scratch_types=[
            pltpu.VMEM((B_PER_W,), jnp.int32),
            pltpu.VMEM((B_PER_W, EMB_K), jnp.float32),
            pltpu.SemaphoreType.DMA,
        ],
        compiler_params=pltpu.CompilerParams(
            use_tc_tiling_on_sc=False, needs_layout_passes=False),
    )


PAIR_BLK = 2048
LT_HALF = PAIR_BLK // 128


def _mlp_body(u2_ref, v2_ref, w1u_ref, w1v_ref, b1_ref, w2_ref,
              out_ref, u4_ref, v4_ref):
    w1u = w1u_ref[...]
    w1v = w1v_ref[...]
    b1 = b1_ref[...]
    w2 = w2_ref[...]
    up = u2_ref[...]
    vp = v2_ref[...]
    for half in range(2):
        u = up[:, EMB_K * half:EMB_K * (half + 1)]
        v = vp[:, EMB_K * half:EMB_K * (half + 1)]
        h = (lax.dot_general(u, w1u, (((1,), (1,)), ((), ())),
                             preferred_element_type=jnp.float32)
             + lax.dot_general(v, w1v, (((1,), (1,)), ((), ())),
                               preferred_element_type=jnp.float32)
             + b1)
        h = jnp.maximum(h, 0.0)
        logit = jnp.sum(h * w2, axis=1)
        sig = jax.nn.sigmoid(logit)
        for gk in range(PAIR_BLK // 512):
            out_ref[0, pl.ds(1024 * gk + 512 * half, 512)] = (
                sig[512 * gk:512 * (gk + 1)])
        ut = u.T
        vt = v.T
        for tt in range(LT_HALF):
            t = 8 * (tt // 4) + 4 * half + (tt % 4)
            u4_ref[:, t] = ut[:, 128 * tt:128 * (tt + 1)].reshape(NBANDS, 8, 128)
            v4_ref[:, t] = vt[:, 128 * tt:128 * (tt + 1)].reshape(NBANDS, 8, 128)


def _mlp(u2, v2, w1u, w1v, b1, w2):
    grid = (NPAIR // PAIR_BLK,)
    return pl.pallas_call(
        _mlp_body,
        grid=grid,
        in_specs=[
            pl.BlockSpec((PAIR_BLK, 128), lambda i: (i, 0)),
            pl.BlockSpec((PAIR_BLK, 128), lambda i: (i, 0)),
            pl.BlockSpec((EMB_K, EMB_K), lambda i: (0, 0)),
            pl.BlockSpec((EMB_K, EMB_K), lambda i: (0, 0)),
            pl.BlockSpec((1, EMB_K), lambda i: (0, 0)),
            pl.BlockSpec((1, EMB_K), lambda i: (0, 0)),
        ],
        out_specs=[
            pl.BlockSpec((1, 2 * PAIR_BLK), lambda i: (0, i)),
            pl.BlockSpec((NBANDS, 2 * LT_HALF, 8, 128), lambda i: (0, i, 0, 0)),
            pl.BlockSpec((NBANDS, 2 * LT_HALF, 8, 128), lambda i: (0, i, 0, 0)),
        ],
        out_shape=[
            jax.ShapeDtypeStruct((1, BATCH), jnp.float32),
            jax.ShapeDtypeStruct((NBANDS, BATCH // 128, 8, 128), jnp.float32),
            jax.ShapeDtypeStruct((NBANDS, BATCH // 128, 8, 128), jnp.float32),
        ],
    )(u2, v2, w1u, w1v, b1, w2)


def kernel(x, W_table, H_table, W1, b1, W2):
    u_idx = x[:, 0]
    v_idx = x[:, 1]
    g = _sc_gather()
    u2 = g(u_idx, W_table)
    v2 = g(v_idx, H_table)
    w1u = W1[:, :EMB_K]
    w1v = W1[:, EMB_K:]
    out2d, u4, v4 = _mlp(u2, v2, w1u, w1v, b1.reshape(1, EMB_K), W2)
    u_emb = u4.transpose(0, 2, 1, 3).reshape(EMB_K, BATCH).T
    v_emb = v4.transpose(0, 2, 1, 3).reshape(EMB_K, BATCH).T
    return (out2d.reshape(BATCH), u_emb, v_emb)
